# speedup vs baseline: 4.2472x; 4.2472x over previous
import functools

import jax
import jax.numpy as jnp
from jax import lax
from jax.experimental import pallas as pl
from jax.experimental.pallas import tpu as pltpu
from jax.experimental.pallas import tpu_sc as plsc

D_SUB = 256
D_MODEL = 1024
NC, NS = 2, 16
NW = NC * NS
B_TOTAL = 4 * 8192
PER_W = B_TOTAL // NW
C = 64
NCHUNK = PER_W // C


def _sc_body(tpos, cpos, epos, dpos, tt, ct, et, dt_, out,
             idx0, idx1, idx2, idx3, r0, r1, r2, r3, sem):
    wid = lax.axis_index("s") * NC + lax.axis_index("c")
    pos_refs = (tpos, cpos, epos, dpos)
    tab_refs = (tt, ct, et, dt_)
    idx_refs = (idx0, idx1, idx2, idx3)
    row_refs = (r0, r1, r2, r3)

    @pl.loop(0, NCHUNK)
    def _chunk(i):
        base = wid * PER_W + i * C
        for t in range(4):
            pltpu.sync_copy(pos_refs[t].at[pl.ds(base, C)], idx_refs[t])
        copies = [
            pltpu.async_copy(tab_refs[t].at[idx_refs[t]], row_refs[t], sem)
            for t in range(4)
        ]
        for cp in copies:
            cp.wait()
        for t in range(4):
            pltpu.sync_copy(
                row_refs[t], out.at[pl.ds(base, C), pl.ds(t * D_SUB, D_SUB)])


@jax.jit
def _lookup(tpos, cpos, epos, dpos, tt, ct, et, dt_):
    mesh = plsc.VectorSubcoreMesh(
        core_axis_name="c", subcore_axis_name="s",
        num_cores=NC, num_subcores=NS)
    scratch = (
        [pltpu.VMEM((C,), jnp.int32) for _ in range(4)]
        + [pltpu.VMEM((C, D_SUB), jnp.float32) for _ in range(4)]
        + [pltpu.SemaphoreType.DMA]
    )
    f = pl.kernel(
        _sc_body,
        out_type=jax.ShapeDtypeStruct((B_TOTAL, D_MODEL), jnp.float32),
        mesh=mesh,
        scratch_types=scratch,
    )
    return f(tpos.reshape(-1), cpos.reshape(-1), epos.reshape(-1),
             dpos.reshape(-1), tt, ct, et, dt_)


def kernel(temporal_pos, causal_depth, epistemic_pos, deontic_pos,
           temporal_table, causal_table, epistemic_table, deontic_table):
    b, s = temporal_pos.shape
    out = _lookup(temporal_pos, causal_depth, epistemic_pos, deontic_pos,
                  temporal_table, causal_table, epistemic_table, deontic_table)
    return out.reshape(b, s, D_MODEL)


# speedup vs baseline: 5.4438x; 1.2817x over previous
import jax
import jax.numpy as jnp
from jax import lax
from jax.experimental import pallas as pl
from jax.experimental.pallas import tpu as pltpu
from jax.experimental.pallas import tpu_sc as plsc

D_SUB = 256
D_MODEL = 1024
NC, NS = 2, 16
NW = NC * NS
B_TOTAL = 4 * 8192
PER_W = B_TOTAL // NW
C = 64
NCHUNK = PER_W // C


def _sc_body(tpos, cpos, epos, dpos, tt, ct, et, dt_, out,
             ia0, ia1, ia2, ia3, b0, b1, b2, b3,
             g0, g1, g2, g3, w0, w1, w2, w3):
    wid = lax.axis_index("s") * NC + lax.axis_index("c")
    base_w = wid * PER_W
    pos_refs = (tpos, cpos, epos, dpos)
    tab_refs = (tt, ct, et, dt_)
    idx_all = (ia0, ia1, ia2, ia3)
    bufs = (b0, b1, b2, b3)
    gsems = (g0, g1, g2, g3)
    wsems = (w0, w1, w2, w3)

    for t in range(4):
        pltpu.sync_copy(pos_refs[t].at[pl.ds(base_w, PER_W)], idx_all[t])

    def gather(chunk, t):
        return pltpu.make_async_copy(
            tab_refs[t].at[idx_all[t].at[pl.ds(chunk * C, C)]],
            bufs[t], gsems[t])

    def write(chunk, t):
        return pltpu.make_async_copy(
            bufs[t],
            out.at[pl.ds(base_w + chunk * C, C), pl.ds(t * D_SUB, D_SUB)],
            wsems[t])

    for t in range(4):
        gather(0, t).start()

    @pl.loop(0, NCHUNK - 1)
    def _steady(chunk):
        for t in range(4):
            gather(chunk, t).wait()
            wd = write(chunk, t)
            wd.start()
            wd.wait()
            gather(chunk + 1, t).start()

    for t in range(4):
        gather(NCHUNK - 1, t).wait()
        wd = write(NCHUNK - 1, t)
        wd.start()
        wd.wait()


@jax.jit
def _lookup(tpos, cpos, epos, dpos, tt, ct, et, dt_):
    mesh = plsc.VectorSubcoreMesh(
        core_axis_name="c", subcore_axis_name="s",
        num_cores=NC, num_subcores=NS)
    scratch = (
        [pltpu.VMEM((PER_W,), jnp.int32) for _ in range(4)]
        + [pltpu.VMEM((C, D_SUB), jnp.float32) for _ in range(4)]
        + [pltpu.SemaphoreType.DMA for _ in range(8)]
    )
    f = pl.kernel(
        _sc_body,
        out_type=jax.ShapeDtypeStruct((B_TOTAL, D_MODEL), jnp.float32),
        mesh=mesh,
        scratch_types=scratch,
    )
    return f(tpos.reshape(-1), cpos.reshape(-1), epos.reshape(-1),
             dpos.reshape(-1), tt, ct, et, dt_)


def kernel(temporal_pos, causal_depth, epistemic_pos, deontic_pos,
           temporal_table, causal_table, epistemic_table, deontic_table):
    b, s = temporal_pos.shape
    out = _lookup(temporal_pos, causal_depth, epistemic_pos, deontic_pos,
                  temporal_table, causal_table, epistemic_table, deontic_table)
    return out.reshape(b, s, D_MODEL)
